# baseline (speedup 1.0000x reference)
import jax
import jax.numpy as jnp
from jax.experimental import pallas as pl
from jax.experimental.pallas import tpu as pltpu

_N = 8192
_D = 64
_CUTOFF = 0.5
_MU = 1.0
_SIGMA = 0.5
_C2 = -1.4426950408889634 / (2.0 * _SIGMA * _SIGMA)

_BM = 128


def _msg_kernel(z_ref, w_ref, b_ref, out_ref):
    out_ref[...] = (
        jnp.dot(z_ref[...], w_ref[...], preferred_element_type=jnp.float32)
        + b_ref[...]
    ).astype(jnp.bfloat16)


def _interact_kernel(dist_ref, msg_ref, out_ref):
    r = dist_ref[...]
    u = 1.0 / r
    t = u - 1.0 / _MU
    w = jnp.where(r < _CUTOFF, jnp.exp2(_C2 * (t * t)), 0.0).astype(jnp.bfloat16)
    out_ref[...] = jnp.dot(w, msg_ref[...], preferred_element_type=jnp.float32)


def kernel(z, dist_matrix, W, B):
    msg = pl.pallas_call(
        _msg_kernel,
        out_shape=jax.ShapeDtypeStruct((_N, _D), jnp.bfloat16),
    )(z, W, B.reshape(1, _D))

    out = pl.pallas_call(
        _interact_kernel,
        grid=(_N // _BM,),
        in_specs=[
            pl.BlockSpec((_BM, _N), lambda i: (i, 0)),
            pl.BlockSpec((_N, _D), lambda i: (0, 0)),
        ],
        out_specs=pl.BlockSpec((_BM, _D), lambda i: (i, 0)),
        out_shape=jax.ShapeDtypeStruct((_N, _D), jnp.float32),
        compiler_params=pltpu.CompilerParams(
            dimension_semantics=("parallel",),
        ),
    )(dist_matrix, msg)
    return out


# speedup vs baseline: 1.2705x; 1.2705x over previous
import jax
import jax.numpy as jnp
from jax.experimental import pallas as pl
from jax.experimental.pallas import tpu as pltpu

_N = 8192
_D = 64
_CUTOFF = 0.5
_MU = 1.0
_SIGMA = 0.5
_C2 = -1.4426950408889634 / (2.0 * _SIGMA * _SIGMA)

_BM = 512


def _msg_kernel(z_ref, w_ref, b_ref, out_ref):
    out_ref[...] = (
        jnp.dot(z_ref[...], w_ref[...], preferred_element_type=jnp.float32)
        + b_ref[...]
    ).astype(jnp.bfloat16)


def _interact_kernel(dist_ref, msg_ref, out_ref):
    r = dist_ref[...]
    u = 1.0 / r
    t = u - 1.0 / _MU
    w = jnp.where(r < _CUTOFF, jnp.exp2(_C2 * (t * t)), 0.0).astype(jnp.bfloat16)
    out_ref[...] = jnp.dot(w, msg_ref[...], preferred_element_type=jnp.float32)


def kernel(z, dist_matrix, W, B):
    msg = pl.pallas_call(
        _msg_kernel,
        out_shape=jax.ShapeDtypeStruct((_N, _D), jnp.bfloat16),
    )(z, W, B.reshape(1, _D))

    out = pl.pallas_call(
        _interact_kernel,
        grid=(_N // _BM,),
        in_specs=[
            pl.BlockSpec((_BM, _N), lambda i: (i, 0)),
            pl.BlockSpec((_N, _D), lambda i: (0, 0)),
        ],
        out_specs=pl.BlockSpec((_BM, _D), lambda i: (i, 0)),
        out_shape=jax.ShapeDtypeStruct((_N, _D), jnp.float32),
        compiler_params=pltpu.CompilerParams(
            dimension_semantics=("parallel",),
        ),
    )(dist_matrix, msg)
    return out


# speedup vs baseline: 1.3030x; 1.0256x over previous
import jax
import jax.numpy as jnp
from jax.experimental import pallas as pl
from jax.experimental.pallas import tpu as pltpu

_N = 8192
_D = 64
_CUTOFF = 0.5
_MU = 1.0
_SIGMA = 0.5
_C2 = -1.4426950408889634 / (2.0 * _SIGMA * _SIGMA)

_BM = 512


def _interact_kernel(z_ref, w_ref, b_ref, dist_ref, out_ref, msg_ref):
    i = pl.program_id(0)

    @pl.when(i == 0)
    def _compute_msg():
        msg_ref[...] = (
            jnp.dot(z_ref[...], w_ref[...], preferred_element_type=jnp.float32)
            + b_ref[...]
        ).astype(jnp.bfloat16)

    r = dist_ref[...]
    u = 1.0 / r
    t = u - 1.0 / _MU
    w = jnp.where(r < _CUTOFF, jnp.exp2(_C2 * (t * t)), 0.0).astype(jnp.bfloat16)
    out_ref[...] = jnp.dot(w, msg_ref[...], preferred_element_type=jnp.float32)


def kernel(z, dist_matrix, W, B):
    out = pl.pallas_call(
        _interact_kernel,
        grid=(_N // _BM,),
        in_specs=[
            pl.BlockSpec((_N, _D), lambda i: (0, 0)),
            pl.BlockSpec((_D, _D), lambda i: (0, 0)),
            pl.BlockSpec((1, _D), lambda i: (0, 0)),
            pl.BlockSpec((_BM, _N), lambda i: (i, 0)),
        ],
        out_specs=pl.BlockSpec((_BM, _D), lambda i: (i, 0)),
        out_shape=jax.ShapeDtypeStruct((_N, _D), jnp.float32),
        scratch_shapes=[pltpu.VMEM((_N, _D), jnp.bfloat16)],
        compiler_params=pltpu.CompilerParams(
            dimension_semantics=("arbitrary",),
        ),
    )(z, W, B.reshape(1, _D), dist_matrix)
    return out
